# Initial kernel scaffold; baseline (speedup 1.0000x reference)
import jax
import jax.numpy as jnp
from jax.experimental import pallas as pl


def kernel(x, w_ih, w_hh, b_ih, b_hh, h0):
    raise NotImplementedError("write your pallas kernel here")


# speedup vs baseline: 1.0767x; 1.0767x over previous
import jax
import jax.numpy as jnp
from jax import lax
from jax.experimental import pallas as pl
from jax.experimental.pallas import tpu as pltpu

_UNROLL = 8
_LANE = 128
_TS = 512


def _round_up(x, m):
    return ((x + m - 1) // m) * m


def _make_body(ts, Hp, last_local):
    num_sub = ts // _UNROLL

    def body(x_ref, wih_ref, whh_ref, b_ref, bhn_ref, h0_ref,
             y_ref, hn_ref, h_sc, gx_sc):
        blk = pl.program_id(0)

        @pl.when(blk == 0)
        def _init():
            h_sc[...] = h0_ref[...]

        gx_sc[...] = (jnp.dot(x_ref[...], wih_ref[...],
                              preferred_element_type=jnp.float32)
                      + b_ref[...])

        whh = whh_ref[...]
        bhn = bhn_ref[...]

        def sub(sb, h):
            base = pl.multiple_of(sb * _UNROLL, _UNROLL)
            gx = gx_sc[pl.ds(base, _UNROLL), :]
            for u in range(_UNROLL):
                row = gx[u:u + 1, :]
                gh = jnp.dot(h.astype(jnp.bfloat16), whh,
                             preferred_element_type=jnp.float32)
                a = row[:, 0:2 * Hp] + gh[:, 0:2 * Hp]
                rz = 0.5 * jnp.tanh(0.5 * a) + 0.5
                r = rz[:, 0:Hp]
                z = rz[:, Hp:2 * Hp]
                n = jnp.tanh(row[:, 2 * Hp:] + r * (gh[:, 2 * Hp:] + bhn))
                h = n + z * (h - n)
                y_ref[pl.ds(base + u, 1), :] = h
            return h

        h_fin = lax.fori_loop(0, num_sub, sub, h_sc[...])
        h_sc[...] = h_fin

        @pl.when(blk == pl.num_programs(0) - 1)
        def _final():
            hn_ref[...] = y_ref[pl.ds(last_local, 1), :]

    return body


def kernel(x, w_ih, w_hh, b_ih, b_hh, h0):
    S, I = x.shape
    H = h0.shape[1]
    Hp = _round_up(H, _LANE)

    def pad_cols(w):
        return jnp.pad(w, ((0, 0), (0, Hp - H)))

    wih_cat = jnp.concatenate(
        [pad_cols(w_ih[g * H:(g + 1) * H].T) for g in range(3)], axis=1)
    whh_cat = jnp.concatenate(
        [jnp.pad(w_hh[g * H:(g + 1) * H].T, ((0, Hp - H), (0, Hp - H)))
         for g in range(3)], axis=1)

    def pad_vec(v):
        return jnp.pad(v.reshape(1, H), ((0, 0), (0, Hp - H)))

    b_cat = jnp.concatenate([pad_vec(b_ih[0:H] + b_hh[0:H]),
                             pad_vec(b_ih[H:2 * H] + b_hh[H:2 * H]),
                             pad_vec(b_ih[2 * H:3 * H])], axis=1)
    bhn = pad_vec(b_hh[2 * H:3 * H])
    h0p = jnp.pad(h0.astype(jnp.float32), ((0, 0), (0, Hp - H)))

    x_c = x.astype(jnp.bfloat16)
    wih_cat = wih_cat.astype(jnp.bfloat16)
    whh_cat = whh_cat.astype(jnp.bfloat16)

    ts = min(_TS, _round_up(S, _UNROLL))
    nblk = -(-S // ts)
    s_pad = nblk * ts
    if s_pad != S:
        x_c = jnp.pad(x_c, ((0, s_pad - S), (0, 0)))
    last_local = (S - 1) - (nblk - 1) * ts

    y_pad, h_n = pl.pallas_call(
        _make_body(ts, Hp, last_local),
        out_shape=(jax.ShapeDtypeStruct((s_pad, Hp), jnp.float32),
                   jax.ShapeDtypeStruct((1, Hp), jnp.float32)),
        grid=(nblk,),
        in_specs=[
            pl.BlockSpec((ts, I), lambda i: (i, 0)),
            pl.BlockSpec((I, 3 * Hp), lambda i: (0, 0)),
            pl.BlockSpec((Hp, 3 * Hp), lambda i: (0, 0)),
            pl.BlockSpec((1, 3 * Hp), lambda i: (0, 0)),
            pl.BlockSpec((1, Hp), lambda i: (0, 0)),
            pl.BlockSpec((1, Hp), lambda i: (0, 0)),
        ],
        out_specs=(
            pl.BlockSpec((ts, Hp), lambda i: (i, 0)),
            pl.BlockSpec((1, Hp), lambda i: (0, 0)),
        ),
        scratch_shapes=[
            pltpu.VMEM((1, Hp), jnp.float32),
            pltpu.VMEM((ts, 3 * Hp), jnp.float32),
        ],
        compiler_params=pltpu.CompilerParams(
            dimension_semantics=("arbitrary",),
            vmem_limit_bytes=48 << 20,
        ),
    )(x_c, wih_cat, whh_cat, b_cat, bhn, h0p)

    return y_pad[:S, :H], h_n[:, :H]
